# baseline (speedup 1.0000x reference)
import functools

import jax
import jax.numpy as jnp
from jax import lax
from jax.experimental import pallas as pl
from jax.experimental.pallas import tpu as pltpu
from jax.experimental.pallas import tpu_sc as plsc

LANES = 128
PADW = 128


def _worker_id():
    info = plsc.get_sparse_core_info()
    return lax.axis_index("s") * info.num_cores + lax.axis_index("c")


@functools.cache
def _make_retile(V: int, D: int):
    info = plsc.get_sparse_core_info()
    NW = info.num_cores * info.num_subcores
    nfull = V // LANES
    rem = V - nfull * LANES
    base_blocks = nfull // NW
    extra = nfull - base_blocks * NW
    assert base_blocks % 2 == 0 and extra % 2 == 0

    mesh = plsc.VectorSubcoreMesh(core_axis_name="c", subcore_axis_name="s")

    @functools.partial(
        pl.kernel,
        out_type=jax.ShapeDtypeStruct((V, PADW), jnp.float32),
        mesh=mesh,
        compiler_params=pltpu.CompilerParams(use_tc_tiling_on_sc=True,
                                             needs_layout_passes=False),
        scratch_types=[
            pltpu.VMEM((2, D, LANES), jnp.float32),
            pltpu.VMEM((2, LANES, PADW), jnp.float32),
            pltpu.SemaphoreType.DMA((2,)),
            pltpu.SemaphoreType.DMA((2,)),
        ],
    )
    def retile_kernel(tblT_hbm, tail_hbm, out_hbm, ibuf, obuf, rsem, wsem):
        wid = _worker_id()
        ex = jnp.minimum(wid, extra // 2)
        blk0 = base_blocks * wid + 2 * ex
        nblk = base_blocks + 2 * jnp.where(wid < extra // 2, 1, 0)
        d_iota = lax.iota(jnp.int32, 16)

        def rstart(j, b):
            pltpu.async_copy(tblT_hbm.at[pl.ds(0, D), pl.ds(j * LANES, LANES)],
                             ibuf.at[b], rsem.at[b])

        def rwait(b):
            pltpu.make_async_copy(
                tblT_hbm.at[pl.ds(0, D), pl.ds(0, LANES)],
                ibuf.at[b], rsem.at[b]).wait()

        def wstart(j, b):
            pltpu.async_copy(obuf.at[b],
                             out_hbm.at[pl.ds(j * LANES, LANES)],
                             wsem.at[b])

        def wwait(b):
            pltpu.make_async_copy(
                obuf.at[b],
                out_hbm.at[pl.ds(0, LANES)],
                wsem.at[b]).wait()

        def transpose(b):
            def grp(i, _):
                for cc in range(8):
                    c = i * 8 + cc
                    for dg in range(D // 16):
                        x = plsc.load_gather(
                            ibuf.at[b],
                            [d_iota + dg * 16, jnp.full((16,), c, jnp.int32)])
                        obuf[b, c, pl.ds(dg * 16, 16)] = x
                return 0
            lax.fori_loop(0, LANES // 8, grp, 0, unroll=False)

        rstart(blk0, 0)
        rstart(blk0 + 1, 1)

        def step(i, _):
            j0 = blk0 + 2 * i
            for b in range(2):
                rwait(b)
                transpose(b)

                @pl.when(i > 0)
                def _():
                    wwait(b)

                wstart(j0 + b, b)

                @pl.when(j0 + 2 + b < blk0 + nblk)
                def _():
                    rstart(j0 + 2 + b, b)
            return 0

        lax.fori_loop(0, nblk // 2, step, 0)
        wwait(0)
        wwait(1)

        if rem:
            @pl.when(wid == NW - 1)
            def _():
                pltpu.sync_copy(tail_hbm, ibuf.at[0, pl.ds(0, rem)])
                pltpu.sync_copy(ibuf.at[0, pl.ds(0, rem)],
                                out_hbm.at[pl.ds(nfull * LANES, rem)])

    return retile_kernel


@functools.cache
def _make_gather(V: int, D: int, SQ: int, ST: int, B: int):
    info = plsc.get_sparse_core_info()
    NW = info.num_cores * info.num_subcores
    assert B // LANES == NW and SQ % 2 == 0 and ST % 2 == 0

    SQ8 = (SQ + 7) // 8
    ST8 = (ST + 7) // 8
    q_row0 = 0
    t_row0 = SQ8 * 8
    n_rows = t_row0 + ST8 * 8

    mesh = plsc.VectorSubcoreMesh(core_axis_name="c", subcore_axis_name="s")

    @functools.partial(
        pl.kernel,
        out_type=(
            jax.ShapeDtypeStruct((SQ, D, B), jnp.float32),
            jax.ShapeDtypeStruct((ST, D, B), jnp.float32),
        ),
        mesh=mesh,
        compiler_params=pltpu.CompilerParams(use_tc_tiling_on_sc=True,
                                             needs_layout_passes=False),
        scratch_types=[
            pltpu.VMEM((n_rows, LANES), jnp.int32),
            pltpu.VMEM((2, LANES, PADW), jnp.float32),
            pltpu.VMEM((2, D, LANES), jnp.float32),
            pltpu.SemaphoreType.DMA((2,)),
            pltpu.SemaphoreType.DMA((2,)),
        ],
    )
    def gather_kernel(table_hbm, qT_hbm, tT_hbm, out_q, out_t,
                      idx_v, gbuf, tbuf, gsem, wsem):
        wid = _worker_id()
        col0 = wid * LANES
        tok_iota = lax.iota(jnp.int32, 16)

        for arr, row0, S8 in ((qT_hbm, q_row0, SQ8), (tT_hbm, t_row0, ST8)):
            for st in range(S8):
                pltpu.sync_copy(
                    arr.at[pl.ds(8 * st, 8), pl.ds(col0, LANES)],
                    idx_v.at[pl.ds(row0 + 8 * st, 8)])

        def transpose(b):
            def grp(i, _):
                for dd in range(8):
                    d = i * 8 + dd
                    for tg in range(LANES // 16):
                        x = plsc.load_gather(
                            gbuf.at[b],
                            [tok_iota + tg * 16, jnp.full((16,), d, jnp.int32)])
                        tbuf[b, d, pl.ds(tg * 16, 16)] = x
                return 0
            lax.fori_loop(0, D // 8, grp, 0, unroll=False)

        def run(out_ref, row0, n_units):
            def gstart(u, b):
                pltpu.async_copy(table_hbm.at[idx_v.at[row0 + u]],
                                 gbuf.at[b], gsem.at[b])

            def gwait(b):
                pltpu.make_async_copy(table_hbm.at[idx_v.at[0]],
                                      gbuf.at[b], gsem.at[b]).wait()

            def wstart(u, b):
                pltpu.async_copy(
                    tbuf.at[b],
                    out_ref.at[u, pl.ds(0, D), pl.ds(col0, LANES)],
                    wsem.at[b])

            def wwait(b):
                pltpu.make_async_copy(
                    tbuf.at[b],
                    out_ref.at[0, pl.ds(0, D), pl.ds(0, LANES)],
                    wsem.at[b]).wait()

            gstart(0, 0)
            gstart(1, 1)

            def step(i, _):
                u0 = 2 * i
                for b in range(2):
                    gwait(b)

                    @pl.when(i > 0)
                    def _():
                        wwait(b)

                    transpose(b)
                    wstart(u0 + b, b)

                    @pl.when(u0 + 2 + b < n_units)
                    def _():
                        gstart(u0 + 2 + b, b)
                return 0

            lax.fori_loop(0, n_units // 2, step, 0)
            wwait(0)
            wwait(1)

        run(out_q, q_row0, SQ)
        run(out_t, t_row0, ST)

    return gather_kernel


def kernel(table, query, title):
    V, D = table.shape
    B, SQ = query.shape
    _, ST = title.shape
    rem = V % LANES
    tblT = jnp.transpose(table)
    tail = jnp.pad(table[V - rem:, :], ((0, 0), (0, PADW - D)))
    qT = jnp.transpose(query.astype(jnp.int32))
    tT = jnp.transpose(title.astype(jnp.int32))
    qT = jnp.pad(qT, ((0, -SQ % 8), (0, 0)))
    tT = jnp.pad(tT, ((0, -ST % 8), (0, 0)))
    table128 = _make_retile(V, D)(tblT, tail)
    out_qT, out_tT = _make_gather(V, D, SQ, ST, B)(table128, qT, tT)
    return (jnp.transpose(out_qT, (2, 0, 1)), jnp.transpose(out_tT, (2, 0, 1)))


# speedup vs baseline: 1.2358x; 1.2358x over previous
import functools

import jax
import jax.numpy as jnp
from jax import lax
from jax.experimental import pallas as pl
from jax.experimental.pallas import tpu as pltpu
from jax.experimental.pallas import tpu_sc as plsc

LANES = 128
PADW = 128


def _worker_id():
    info = plsc.get_sparse_core_info()
    return lax.axis_index("s") * info.num_cores + lax.axis_index("c")


@functools.cache
def _make_retile(V: int, D: int):
    info = plsc.get_sparse_core_info()
    NW = info.num_cores * info.num_subcores
    nfull = V // LANES
    rem = V - nfull * LANES
    base_blocks = nfull // NW
    extra = nfull - base_blocks * NW
    assert base_blocks % 2 == 0 and extra % 2 == 0

    mesh = plsc.VectorSubcoreMesh(core_axis_name="c", subcore_axis_name="s")

    @functools.partial(
        pl.kernel,
        out_type=jax.ShapeDtypeStruct((V, PADW), jnp.float32),
        mesh=mesh,
        compiler_params=pltpu.CompilerParams(use_tc_tiling_on_sc=True,
                                             needs_layout_passes=False),
        scratch_types=[
            pltpu.VMEM((2, D, LANES), jnp.float32),
            pltpu.VMEM((2, LANES, PADW), jnp.float32),
            pltpu.SemaphoreType.DMA((2,)),
            pltpu.SemaphoreType.DMA((2,)),
        ],
    )
    def retile_kernel(tblT_hbm, tail_hbm, out_hbm, ibuf, obuf, rsem, wsem):
        wid = _worker_id()
        ex = jnp.minimum(wid, extra // 2)
        blk0 = base_blocks * wid + 2 * ex
        nblk = base_blocks + 2 * jnp.where(wid < extra // 2, 1, 0)
        d_iota = lax.iota(jnp.int32, 16)

        def rstart(j, b):
            pltpu.async_copy(tblT_hbm.at[pl.ds(0, D), pl.ds(j * LANES, LANES)],
                             ibuf.at[b], rsem.at[b])

        def rwait(b):
            pltpu.make_async_copy(
                tblT_hbm.at[pl.ds(0, D), pl.ds(0, LANES)],
                ibuf.at[b], rsem.at[b]).wait()

        def wstart(j, b):
            pltpu.async_copy(obuf.at[b],
                             out_hbm.at[pl.ds(j * LANES, LANES)],
                             wsem.at[b])

        def wwait(b):
            pltpu.make_async_copy(
                obuf.at[b],
                out_hbm.at[pl.ds(0, LANES)],
                wsem.at[b]).wait()

        c_groups = [d_iota + 16 * g for g in range(LANES // 16)]

        def transpose(b):
            def grp(i, _):
                for dd in range(2):
                    d = i * 2 + dd
                    dv = jnp.full((16,), d, jnp.int32)
                    for cg in range(LANES // 16):
                        x = ibuf[b, d, pl.ds(cg * 16, 16)]
                        plsc.store_scatter(obuf.at[b], [c_groups[cg], dv], x)
                return 0
            lax.fori_loop(0, D // 2, grp, 0, unroll=False)

        rstart(blk0, 0)
        rstart(blk0 + 1, 1)

        def step(i, _):
            j0 = blk0 + 2 * i
            for b in range(2):
                rwait(b)
                transpose(b)

                @pl.when(i > 0)
                def _():
                    wwait(b)

                wstart(j0 + b, b)

                @pl.when(j0 + 2 + b < blk0 + nblk)
                def _():
                    rstart(j0 + 2 + b, b)
            return 0

        lax.fori_loop(0, nblk // 2, step, 0)
        wwait(0)
        wwait(1)

        if rem:
            @pl.when(wid == NW - 1)
            def _():
                pltpu.sync_copy(tail_hbm, ibuf.at[0, pl.ds(0, rem)])
                pltpu.sync_copy(ibuf.at[0, pl.ds(0, rem)],
                                out_hbm.at[pl.ds(nfull * LANES, rem)])

    return retile_kernel


@functools.cache
def _make_gather(V: int, D: int, SQ: int, ST: int, B: int):
    info = plsc.get_sparse_core_info()
    NW = info.num_cores * info.num_subcores
    assert B // LANES == NW and SQ % 2 == 0 and ST % 2 == 0

    SQ8 = (SQ + 7) // 8
    ST8 = (ST + 7) // 8
    q_row0 = 0
    t_row0 = SQ8 * 8
    n_rows = t_row0 + ST8 * 8

    mesh = plsc.VectorSubcoreMesh(core_axis_name="c", subcore_axis_name="s")

    @functools.partial(
        pl.kernel,
        out_type=(
            jax.ShapeDtypeStruct((SQ, D, B), jnp.float32),
            jax.ShapeDtypeStruct((ST, D, B), jnp.float32),
        ),
        mesh=mesh,
        compiler_params=pltpu.CompilerParams(use_tc_tiling_on_sc=True,
                                             needs_layout_passes=False),
        scratch_types=[
            pltpu.VMEM((n_rows, LANES), jnp.int32),
            pltpu.VMEM((2, LANES, PADW), jnp.float32),
            pltpu.VMEM((2, D, LANES), jnp.float32),
            pltpu.SemaphoreType.DMA((2,)),
            pltpu.SemaphoreType.DMA((2,)),
        ],
    )
    def gather_kernel(table_hbm, qT_hbm, tT_hbm, out_q, out_t,
                      idx_v, gbuf, tbuf, gsem, wsem):
        wid = _worker_id()
        col0 = wid * LANES
        tok_iota = lax.iota(jnp.int32, 16)

        for arr, row0, S8 in ((qT_hbm, q_row0, SQ8), (tT_hbm, t_row0, ST8)):
            for st in range(S8):
                pltpu.sync_copy(
                    arr.at[pl.ds(8 * st, 8), pl.ds(col0, LANES)],
                    idx_v.at[pl.ds(row0 + 8 * st, 8)])

        d_groups = [tok_iota + 16 * g for g in range(D // 16)]

        def transpose(b):
            def grp(i, _):
                for tt in range(4):
                    t = i * 4 + tt
                    tv = jnp.full((16,), t, jnp.int32)
                    for dg in range(D // 16):
                        x = gbuf[b, t, pl.ds(dg * 16, 16)]
                        plsc.store_scatter(tbuf.at[b], [d_groups[dg], tv], x)
                return 0
            lax.fori_loop(0, LANES // 4, grp, 0, unroll=False)

        def run(out_ref, row0, n_units):
            def gstart(u, b):
                pltpu.async_copy(table_hbm.at[idx_v.at[row0 + u]],
                                 gbuf.at[b], gsem.at[b])

            def gwait(b):
                pltpu.make_async_copy(table_hbm.at[idx_v.at[0]],
                                      gbuf.at[b], gsem.at[b]).wait()

            def wstart(u, b):
                pltpu.async_copy(
                    tbuf.at[b],
                    out_ref.at[u, pl.ds(0, D), pl.ds(col0, LANES)],
                    wsem.at[b])

            def wwait(b):
                pltpu.make_async_copy(
                    tbuf.at[b],
                    out_ref.at[0, pl.ds(0, D), pl.ds(0, LANES)],
                    wsem.at[b]).wait()

            gstart(0, 0)
            gstart(1, 1)

            def step(i, _):
                u0 = 2 * i
                for b in range(2):
                    gwait(b)

                    @pl.when(i > 0)
                    def _():
                        wwait(b)

                    transpose(b)
                    wstart(u0 + b, b)

                    @pl.when(u0 + 2 + b < n_units)
                    def _():
                        gstart(u0 + 2 + b, b)
                return 0

            lax.fori_loop(0, n_units // 2, step, 0)
            wwait(0)
            wwait(1)

        run(out_q, q_row0, SQ)
        run(out_t, t_row0, ST)

    return gather_kernel


def kernel(table, query, title):
    V, D = table.shape
    B, SQ = query.shape
    _, ST = title.shape
    rem = V % LANES
    tblT = jnp.transpose(table)
    tail = jnp.pad(table[V - rem:, :], ((0, 0), (0, PADW - D)))
    qT = jnp.transpose(query.astype(jnp.int32))
    tT = jnp.transpose(title.astype(jnp.int32))
    qT = jnp.pad(qT, ((0, -SQ % 8), (0, 0)))
    tT = jnp.pad(tT, ((0, -ST % 8), (0, 0)))
    table128 = _make_retile(V, D)(tblT, tail)
    out_qT, out_tT = _make_gather(V, D, SQ, ST, B)(table128, qT, tT)
    return (jnp.transpose(out_qT, (2, 0, 1)), jnp.transpose(out_tT, (2, 0, 1)))


# speedup vs baseline: 1.2469x; 1.0090x over previous
import functools

import jax
import jax.numpy as jnp
from jax import lax
from jax.experimental import pallas as pl
from jax.experimental.pallas import tpu as pltpu
from jax.experimental.pallas import tpu_sc as plsc

LANES = 128
PADW = 128


def _worker_id():
    info = plsc.get_sparse_core_info()
    return lax.axis_index("s") * info.num_cores + lax.axis_index("c")


@functools.cache
def _make_retile(V: int, D: int):
    info = plsc.get_sparse_core_info()
    NW = info.num_cores * info.num_subcores
    nfull = V // LANES
    rem = V - nfull * LANES
    base_blocks = nfull // NW
    extra = nfull - base_blocks * NW
    assert base_blocks % 2 == 0 and extra % 2 == 0

    mesh = plsc.VectorSubcoreMesh(core_axis_name="c", subcore_axis_name="s")

    @functools.partial(
        pl.kernel,
        out_type=jax.ShapeDtypeStruct((V, PADW), jnp.float32),
        mesh=mesh,
        compiler_params=pltpu.CompilerParams(use_tc_tiling_on_sc=True,
                                             needs_layout_passes=False),
        scratch_types=[
            pltpu.VMEM((2, D, LANES), jnp.float32),
            pltpu.VMEM((2, LANES, PADW), jnp.float32),
            pltpu.SemaphoreType.DMA((2,)),
            pltpu.SemaphoreType.DMA((2,)),
        ],
    )
    def retile_kernel(tblT_hbm, tail_hbm, out_hbm, ibuf, obuf, rsem, wsem):
        wid = _worker_id()
        ex = jnp.minimum(wid, extra // 2)
        blk0 = base_blocks * wid + 2 * ex
        nblk = base_blocks + 2 * jnp.where(wid < extra // 2, 1, 0)
        d_iota = lax.iota(jnp.int32, 16)

        def rstart(j, b):
            pltpu.async_copy(tblT_hbm.at[pl.ds(0, D), pl.ds(j * LANES, LANES)],
                             ibuf.at[b], rsem.at[b])

        def rwait(b):
            pltpu.make_async_copy(
                tblT_hbm.at[pl.ds(0, D), pl.ds(0, LANES)],
                ibuf.at[b], rsem.at[b]).wait()

        def wstart(j, b):
            pltpu.async_copy(obuf.at[b],
                             out_hbm.at[pl.ds(j * LANES, LANES)],
                             wsem.at[b])

        def wwait(b):
            pltpu.make_async_copy(
                obuf.at[b],
                out_hbm.at[pl.ds(0, LANES)],
                wsem.at[b]).wait()

        c_groups = [d_iota + 16 * g for g in range(LANES // 16)]

        def transpose(b):
            def grp(i, _):
                for dd in range(2):
                    d = i * 2 + dd
                    dv = jnp.full((16,), d, jnp.int32)
                    xs = [ibuf[b, d, pl.ds(cg * 16, 16)]
                          for cg in range(LANES // 16)]
                    for cg in range(LANES // 16):
                        plsc.store_scatter(obuf.at[b], [c_groups[cg], dv],
                                           xs[cg])
                return 0
            lax.fori_loop(0, D // 2, grp, 0, unroll=False)

        rstart(blk0, 0)
        rstart(blk0 + 1, 1)

        def step(i, _):
            j0 = blk0 + 2 * i
            for b in range(2):
                rwait(b)
                transpose(b)

                @pl.when(i > 0)
                def _():
                    wwait(b)

                wstart(j0 + b, b)

                @pl.when(j0 + 2 + b < blk0 + nblk)
                def _():
                    rstart(j0 + 2 + b, b)
            return 0

        lax.fori_loop(0, nblk // 2, step, 0)
        wwait(0)
        wwait(1)

        if rem:
            @pl.when(wid == NW - 1)
            def _():
                pltpu.sync_copy(tail_hbm, ibuf.at[0, pl.ds(0, rem)])
                pltpu.sync_copy(ibuf.at[0, pl.ds(0, rem)],
                                out_hbm.at[pl.ds(nfull * LANES, rem)])

    return retile_kernel


@functools.cache
def _make_gather(V: int, D: int, SQ: int, ST: int, B: int):
    info = plsc.get_sparse_core_info()
    NW = info.num_cores * info.num_subcores
    assert B // LANES == NW and SQ % 2 == 0 and ST % 2 == 0

    SQ8 = (SQ + 7) // 8
    ST8 = (ST + 7) // 8
    q_row0 = 0
    t_row0 = SQ8 * 8
    n_rows = t_row0 + ST8 * 8

    mesh = plsc.VectorSubcoreMesh(core_axis_name="c", subcore_axis_name="s")

    @functools.partial(
        pl.kernel,
        out_type=(
            jax.ShapeDtypeStruct((SQ, D, B), jnp.float32),
            jax.ShapeDtypeStruct((ST, D, B), jnp.float32),
        ),
        mesh=mesh,
        compiler_params=pltpu.CompilerParams(use_tc_tiling_on_sc=True,
                                             needs_layout_passes=False),
        scratch_types=[
            pltpu.VMEM((n_rows, LANES), jnp.int32),
            pltpu.VMEM((2, LANES, PADW), jnp.float32),
            pltpu.VMEM((2, D, LANES), jnp.float32),
            pltpu.SemaphoreType.DMA((2,)),
            pltpu.SemaphoreType.DMA((2,)),
        ],
    )
    def gather_kernel(table_hbm, qT_hbm, tT_hbm, out_q, out_t,
                      idx_v, gbuf, tbuf, gsem, wsem):
        wid = _worker_id()
        col0 = wid * LANES
        tok_iota = lax.iota(jnp.int32, 16)

        for arr, row0, S8 in ((qT_hbm, q_row0, SQ8), (tT_hbm, t_row0, ST8)):
            for st in range(S8):
                pltpu.sync_copy(
                    arr.at[pl.ds(8 * st, 8), pl.ds(col0, LANES)],
                    idx_v.at[pl.ds(row0 + 8 * st, 8)])

        d_groups = [tok_iota + 16 * g for g in range(D // 16)]

        def transpose(b):
            def grp(i, _):
                for tt in range(2):
                    t0 = i * 4 + tt * 2
                    tvs = [jnp.full((16,), t0 + j, jnp.int32) for j in range(2)]
                    xs = [gbuf[b, t0 + j, pl.ds(dg * 16, 16)]
                          for j in range(2) for dg in range(D // 16)]
                    for j in range(2):
                        for dg in range(D // 16):
                            plsc.store_scatter(tbuf.at[b],
                                               [d_groups[dg], tvs[j]],
                                               xs[j * (D // 16) + dg])
                return 0
            lax.fori_loop(0, LANES // 4, grp, 0, unroll=False)

        def run(out_ref, row0, n_units):
            def gstart(u, b):
                pltpu.async_copy(table_hbm.at[idx_v.at[row0 + u]],
                                 gbuf.at[b], gsem.at[b])

            def gwait(b):
                pltpu.make_async_copy(table_hbm.at[idx_v.at[0]],
                                      gbuf.at[b], gsem.at[b]).wait()

            def wstart(u, b):
                pltpu.async_copy(
                    tbuf.at[b],
                    out_ref.at[u, pl.ds(0, D), pl.ds(col0, LANES)],
                    wsem.at[b])

            def wwait(b):
                pltpu.make_async_copy(
                    tbuf.at[b],
                    out_ref.at[0, pl.ds(0, D), pl.ds(0, LANES)],
                    wsem.at[b]).wait()

            gstart(0, 0)
            gstart(1, 1)

            def step(i, _):
                u0 = 2 * i
                for b in range(2):
                    gwait(b)

                    @pl.when(i > 0)
                    def _():
                        wwait(b)

                    transpose(b)
                    wstart(u0 + b, b)

                    @pl.when(u0 + 2 + b < n_units)
                    def _():
                        gstart(u0 + 2 + b, b)
                return 0

            lax.fori_loop(0, n_units // 2, step, 0)
            wwait(0)
            wwait(1)

        run(out_q, q_row0, SQ)
        run(out_t, t_row0, ST)

    return gather_kernel


def kernel(table, query, title):
    V, D = table.shape
    B, SQ = query.shape
    _, ST = title.shape
    rem = V % LANES
    tblT = jnp.transpose(table)
    tail = jnp.pad(table[V - rem:, :], ((0, 0), (0, PADW - D)))
    qT = jnp.transpose(query.astype(jnp.int32))
    tT = jnp.transpose(title.astype(jnp.int32))
    qT = jnp.pad(qT, ((0, -SQ % 8), (0, 0)))
    tT = jnp.pad(tT, ((0, -ST % 8), (0, 0)))
    table128 = _make_retile(V, D)(tblT, tail)
    out_qT, out_tT = _make_gather(V, D, SQ, ST, B)(table128, qT, tT)
    return (jnp.transpose(out_qT, (2, 0, 1)), jnp.transpose(out_tT, (2, 0, 1)))
